# baseline (speedup 1.0000x reference)
import functools

import jax
import jax.numpy as jnp
from jax import lax
from jax.experimental import pallas as pl
from jax.experimental.pallas import tpu as pltpu
from jax.experimental.pallas import tpu_sc as plsc

ALPHA = 0.05
B, N, D = 4, 8192, 1024
K = N // 2
LANES = 128
ROWS = N // LANES
TOPROWS = K // LANES

NW = 32
RPW = (B * K) // NW
CHUNK = 32
NSTEPS = RPW // CHUNK
WPB = NW // B

MR = 512


def _topk_body(w_ref, gidx_ref, tv_ref, ti_ref):
    v = w_ref[...]
    r = lax.broadcasted_iota(jnp.int32, (B * ROWS, LANES), 0)
    c = lax.broadcasted_iota(jnp.int32, (B * ROWS, LANES), 1)
    lin = (r % ROWS) * LANES + c
    idx = lin

    size = 2
    while size <= N:
        d = size // 2
        while d >= 1:
            if d < LANES:
                ax, s = 1, d
            else:
                ax, s = 0, d // LANES
            low = (lin & d) == 0
            pv = jnp.where(low, jnp.roll(v, -s, axis=ax), jnp.roll(v, s, axis=ax))
            pi = jnp.where(low, jnp.roll(idx, -s, axis=ax), jnp.roll(idx, s, axis=ax))
            first = (v > pv) | ((v == pv) & (idx < pi))
            asc = (lin & size) != 0
            take_self = first ^ (~low) ^ asc
            v = jnp.where(take_self, v, pv)
            idx = jnp.where(take_self, idx, pi)
            d //= 2
        size *= 2

    blocks = []
    for b in range(B):
        blocks.append(idx[b * ROWS:b * ROWS + TOPROWS, :] + b * N)
        at_kth = lin[:ROWS, :] == (K - 1)
        vb = v[b * ROWS:(b + 1) * ROWS, :]
        ib = idx[b * ROWS:(b + 1) * ROWS, :]
        tv_ref[b] = jnp.sum(jnp.where(at_kth, vb, jnp.zeros_like(vb)))
        ti_ref[b] = jnp.sum(jnp.where(at_kth, ib, jnp.zeros_like(ib)))
    gidx_ref[...] = jnp.concatenate(blocks, axis=0)


def _topk_call(w2):
    return pl.pallas_call(
        _topk_body,
        out_shape=[
            jax.ShapeDtypeStruct((B * TOPROWS, LANES), jnp.int32),
            jax.ShapeDtypeStruct((B,), jnp.float32),
            jax.ShapeDtypeStruct((B,), jnp.int32),
        ],
        in_specs=[pl.BlockSpec((B * ROWS, LANES), lambda: (0, 0))],
        out_specs=[
            pl.BlockSpec((B * TOPROWS, LANES), lambda: (0, 0)),
            pl.BlockSpec(memory_space=pltpu.SMEM),
            pl.BlockSpec(memory_space=pltpu.SMEM),
        ],
    )(w2)


def _msum_body(tv_ref, ti_ref, a_ref, seq_ref, o_ref):
    b = pl.program_id(0)
    s = pl.program_id(1)
    t = tv_ref[b]
    bt = ti_ref[b]
    a = a_ref[...]
    j = lax.broadcasted_iota(jnp.int32, (MR, 1), 0) + s * MR
    sel = (a > t) | ((a == t) & (j <= bt))
    m = jnp.where(sel, 0.0, 1.0).astype(jnp.float32)
    part = jnp.sum(seq_ref[0] * m, axis=0, keepdims=True)

    @pl.when(s == 0)
    def _():
        o_ref[...] = jnp.zeros_like(o_ref)

    o_ref[...] += part[None]


def _masked_sum_call(tv, ti, a_col, seq):
    return pl.pallas_call(
        _msum_body,
        out_shape=jax.ShapeDtypeStruct((B, 1, D), jnp.float32),
        grid=(B, N // MR),
        in_specs=[
            pl.BlockSpec(memory_space=pltpu.SMEM),
            pl.BlockSpec(memory_space=pltpu.SMEM),
            pl.BlockSpec((MR, 1), lambda b, s: (b * (N // MR) + s, 0)),
            pl.BlockSpec((1, MR, D), lambda b, s: (b, s, 0)),
        ],
        out_specs=pl.BlockSpec((1, 1, D), lambda b, s: (b, 0, 0)),
        compiler_params=pltpu.CompilerParams(
            dimension_semantics=("parallel", "arbitrary")),
    )(tv, ti, a_col, seq)


def _sc_gather_call(seq_flat, gidx3):
    mesh = plsc.VectorSubcoreMesh(core_axis_name="c", subcore_axis_name="s")

    @functools.partial(
        pl.kernel,
        out_type=jax.ShapeDtypeStruct((B, K + 1, D), jnp.float32),
        mesh=mesh,
        scratch_types=[
            pltpu.VMEM((NSTEPS, CHUNK), jnp.int32),
            pltpu.VMEM((CHUNK, D), jnp.float32),
            pltpu.VMEM((CHUNK, D), jnp.float32),
            pltpu.SemaphoreType.DMA,
            pltpu.SemaphoreType.DMA,
            pltpu.SemaphoreType.DMA,
            pltpu.SemaphoreType.DMA,
        ],
    )
    def k(seq_hbm, gidx_hbm, out_hbm, idx_v, buf0, buf1, g0, g1, w0, w1):
        wid = lax.axis_index("s") * 2 + lax.axis_index("c")
        b = wid // WPB
        chunk_base = (wid % WPB) * RPW

        pltpu.sync_copy(gidx_hbm.at[wid], idx_v)

        def out_at(tt):
            return out_hbm.at[b, pl.ds(chunk_base + tt * CHUNK, CHUNK)]

        bufs = ((buf0, g0, w0), (buf1, g1, w1))

        @pl.loop(0, NSTEPS, step=2)
        def _(t):
            for jj in range(2):
                buf, gs, ws = bufs[jj]
                tt = t + jj

                @pl.when(tt >= 2)
                def _():
                    pltpu.make_async_copy(buf, out_at(tt - 2), ws).wait()

                pltpu.async_copy(seq_hbm.at[idx_v.at[tt]], buf, gs).wait()
                pltpu.async_copy(buf, out_at(tt), ws)

        pltpu.make_async_copy(buf0, out_at(NSTEPS - 2), w0).wait()
        pltpu.make_async_copy(buf1, out_at(NSTEPS - 1), w1).wait()

    return k(seq_flat, gidx3)


def _fin_body(orow_ref, rsum_ref, o_ref):
    cnt = jnp.float32(K) + jnp.float32(1e-10)
    o_ref[...] = jnp.float32(ALPHA) * (rsum_ref[...] / cnt)


def _finalize_call(out1, rsum):
    out_flat = pl.pallas_call(
        _fin_body,
        out_shape=jax.ShapeDtypeStruct((B * (K + 1) * D,), jnp.float32),
        grid=(B,),
        in_specs=[
            pl.BlockSpec((D,), lambda b: (b * (K + 1) + K,)),
            pl.BlockSpec((D,), lambda b: (b,)),
        ],
        out_specs=pl.BlockSpec((D,), lambda b: (b * (K + 1) + K,)),
        input_output_aliases={0: 0},
    )(out1.reshape(B * (K + 1) * D), rsum.reshape(B * D))
    return out_flat.reshape(B, K + 1, D)


def kernel(seq, attn_weights):
    if attn_weights.ndim == 3:
        attn_weights = jnp.squeeze(attn_weights, axis=1)
    w2 = attn_weights.reshape(B * ROWS, LANES)
    gidx, tv, ti = _topk_call(w2)
    rsum = _masked_sum_call(tv, ti, attn_weights.reshape(B * N, 1), seq)
    out1 = _sc_gather_call(
        seq.reshape(B * N, D), gidx.reshape(NW, NSTEPS, CHUNK))
    return _finalize_call(out1, rsum)


# speedup vs baseline: 5.1306x; 5.1306x over previous
import functools

import jax
import jax.numpy as jnp
from jax import lax
from jax.experimental import pallas as pl
from jax.experimental.pallas import tpu as pltpu
from jax.experimental.pallas import tpu_sc as plsc

ALPHA = 0.05
B, N, D = 4, 8192, 1024
K = N // 2
LANES = 128
ROWS = N // LANES
TOPROWS = K // LANES

NW = 32
RPW = (B * K) // NW
CHUNK = 32
NSTEPS = RPW // CHUNK
WPB = NW // B

MR = 512


def _topk_body(w_ref, gidx_ref, tv_ref, ti_ref):
    v = w_ref[...]
    r = lax.broadcasted_iota(jnp.int32, (B * ROWS, LANES), 0)
    c = lax.broadcasted_iota(jnp.int32, (B * ROWS, LANES), 1)
    lin = (r % ROWS) * LANES + c
    idx = lin

    size = 2
    while size <= N:
        d = size // 2
        while d >= 1:
            if d < LANES:
                ax, s = 1, d
            else:
                ax, s = 0, d // LANES
            low = (lin & d) == 0
            pv = jnp.where(low, jnp.roll(v, -s, axis=ax), jnp.roll(v, s, axis=ax))
            pi = jnp.where(low, jnp.roll(idx, -s, axis=ax), jnp.roll(idx, s, axis=ax))
            first = (v > pv) | ((v == pv) & (idx < pi))
            asc = (lin & size) != 0
            take_self = first ^ (~low) ^ asc
            v = jnp.where(take_self, v, pv)
            idx = jnp.where(take_self, idx, pi)
            d //= 2
        size *= 2

    blocks = []
    for b in range(B):
        blocks.append(idx[b * ROWS:b * ROWS + TOPROWS, :] + b * N)
        at_kth = lin[:ROWS, :] == (K - 1)
        vb = v[b * ROWS:(b + 1) * ROWS, :]
        ib = idx[b * ROWS:(b + 1) * ROWS, :]
        tv_ref[b] = jnp.sum(jnp.where(at_kth, vb, jnp.zeros_like(vb)))
        ti_ref[b] = jnp.sum(jnp.where(at_kth, ib, jnp.zeros_like(ib)))
    gidx_ref[...] = jnp.concatenate(blocks, axis=0)


def _topk_call(w2):
    return pl.pallas_call(
        _topk_body,
        out_shape=[
            jax.ShapeDtypeStruct((B * TOPROWS, LANES), jnp.int32),
            jax.ShapeDtypeStruct((B,), jnp.float32),
            jax.ShapeDtypeStruct((B,), jnp.int32),
        ],
        in_specs=[pl.BlockSpec((B * ROWS, LANES), lambda: (0, 0))],
        out_specs=[
            pl.BlockSpec((B * TOPROWS, LANES), lambda: (0, 0)),
            pl.BlockSpec(memory_space=pltpu.SMEM),
            pl.BlockSpec(memory_space=pltpu.SMEM),
        ],
    )(w2)


def _msum_body(tv_ref, ti_ref, a_ref, seq_ref, o_ref):
    b = pl.program_id(0)
    s = pl.program_id(1)
    t = tv_ref[b]
    bt = ti_ref[b]
    a = a_ref[...]
    j = lax.broadcasted_iota(jnp.int32, (MR, 1), 0) + s * MR
    sel = (a > t) | ((a == t) & (j <= bt))
    m = jnp.where(sel, 0.0, 1.0).astype(jnp.float32)
    part = jnp.sum(seq_ref[0] * m, axis=0, keepdims=True)

    @pl.when(s == 0)
    def _():
        o_ref[...] = jnp.zeros_like(o_ref)

    o_ref[...] += part[None]


def _masked_sum_call(tv, ti, a_col, seq):
    return pl.pallas_call(
        _msum_body,
        out_shape=jax.ShapeDtypeStruct((B, 1, D), jnp.float32),
        grid=(B, N // MR),
        in_specs=[
            pl.BlockSpec(memory_space=pltpu.SMEM),
            pl.BlockSpec(memory_space=pltpu.SMEM),
            pl.BlockSpec((MR, 1), lambda b, s: (b * (N // MR) + s, 0)),
            pl.BlockSpec((1, MR, D), lambda b, s: (b, s, 0)),
        ],
        out_specs=pl.BlockSpec((1, 1, D), lambda b, s: (b, 0, 0)),
        compiler_params=pltpu.CompilerParams(
            dimension_semantics=("parallel", "arbitrary")),
    )(tv, ti, a_col, seq)


def _sc_gather_call(seq_flat, gidx3):
    mesh = plsc.VectorSubcoreMesh(core_axis_name="c", subcore_axis_name="s")

    @functools.partial(
        pl.kernel,
        out_type=jax.ShapeDtypeStruct((B, K + 1, D), jnp.float32),
        mesh=mesh,
        scratch_types=[
            pltpu.VMEM((NSTEPS, CHUNK), jnp.int32),
            pltpu.VMEM((CHUNK, D), jnp.float32),
            pltpu.VMEM((CHUNK, D), jnp.float32),
            pltpu.SemaphoreType.DMA,
            pltpu.SemaphoreType.DMA,
            pltpu.SemaphoreType.DMA,
            pltpu.SemaphoreType.DMA,
        ],
    )
    def k(seq_hbm, gidx_hbm, out_hbm, idx_v, buf0, buf1, g0, g1, w0, w1):
        wid = lax.axis_index("s") * 2 + lax.axis_index("c")
        b = wid // WPB
        chunk_base = (wid % WPB) * RPW

        pltpu.sync_copy(gidx_hbm.at[wid], idx_v)

        def out_at(tt):
            return out_hbm.at[b, pl.ds(chunk_base + tt * CHUNK, CHUNK)]

        bufs = ((buf0, g0, w0), (buf1, g1, w1))

        @pl.loop(0, NSTEPS, step=2)
        def _(t):
            for jj in range(2):
                buf, gs, ws = bufs[jj]
                tt = t + jj

                @pl.when(tt >= 2)
                def _():
                    pltpu.make_async_copy(buf, out_at(tt - 2), ws).wait()

                pltpu.async_copy(seq_hbm.at[idx_v.at[tt]], buf, gs).wait()
                pltpu.async_copy(buf, out_at(tt), ws)

        pltpu.make_async_copy(buf0, out_at(NSTEPS - 2), w0).wait()
        pltpu.make_async_copy(buf1, out_at(NSTEPS - 1), w1).wait()

    return k(seq_flat, gidx3)


def _fin_body(orow_ref, rsum_ref, o_ref):
    b = pl.program_id(0)
    cnt = jnp.float32(K) + jnp.float32(1e-10)
    row = jnp.float32(ALPHA) * (rsum_ref[pl.ds(b, 1), :] / cnt)
    o_ref[...] = jnp.broadcast_to(row[:, None, :], o_ref.shape)


def _finalize_call(out1, rsum):
    return pl.pallas_call(
        _fin_body,
        out_shape=jax.ShapeDtypeStruct((B, K + 1, D), jnp.float32),
        grid=(B,),
        in_specs=[
            pl.BlockSpec((1, 8, D), lambda b: (b, K // 8, 0)),
            pl.BlockSpec((B, D), lambda b: (0, 0)),
        ],
        out_specs=pl.BlockSpec((1, 8, D), lambda b: (b, K // 8, 0)),
        input_output_aliases={0: 0},
    )(out1, rsum)


def kernel(seq, attn_weights):
    if attn_weights.ndim == 3:
        attn_weights = jnp.squeeze(attn_weights, axis=1)
    w2 = attn_weights.reshape(B * ROWS, LANES)
    gidx, tv, ti = _topk_call(w2)
    rsum = _masked_sum_call(tv, ti, attn_weights.reshape(B * N, 1), seq)
    out1 = _sc_gather_call(
        seq.reshape(B * N, D), gidx.reshape(NW, NSTEPS, CHUNK))
    return _finalize_call(out1, rsum.reshape(B, D))
